# baseline (speedup 1.0000x reference)
import functools

import jax
import jax.numpy as jnp
from jax import lax
from jax.experimental import pallas as pl
from jax.experimental.pallas import tpu as pltpu
from jax.experimental.pallas import tpu_sc as plsc

BATCH = 16
SEQ_LEN = 1024
TOTAL = BATCH * SEQ_LEN
EMB_DIM = 64
ENC_SIZE = 512
REL_DIM = 256
PAIR = 2 * EMB_DIM

NUM_WORKERS = 32
TPW = TOTAL // NUM_WORKERS
CHUNK = 128
NCHUNK = TPW // CHUNK


def _gather_body(ids_pos, ids_cat, ids_sense, flat_idx,
                 pos_t, cat_t, sense_t, src_enc,
                 out_pos, out_cat, out_sense, out_head,
                 idx_v, p_v, c_v, s_v, h_v, gsem, wsem):
    wid = lax.axis_index("s") * 2 + lax.axis_index("c")
    base = wid * TPW
    staged = []
    for k, ids in enumerate((ids_pos, ids_cat, ids_sense, flat_idx)):
        for j in range(NCHUNK):
            rows = pl.ds(base + j * CHUNK, CHUNK)
            staged.append(
                pltpu.async_copy(ids.at[rows], idx_v.at[k * NCHUNK + j], gsem))
    for h in staged:
        h.wait()
    for j in range(NCHUNK):
        rows = pl.ds(base + j * CHUNK, CHUNK)
        gathers = (
            pltpu.async_copy(pos_t.at[idx_v.at[0 * NCHUNK + j]], p_v, gsem),
            pltpu.async_copy(cat_t.at[idx_v.at[1 * NCHUNK + j]], c_v, gsem),
            pltpu.async_copy(sense_t.at[idx_v.at[2 * NCHUNK + j]], s_v, gsem),
            pltpu.async_copy(src_enc.at[idx_v.at[3 * NCHUNK + j]], h_v, gsem),
        )
        for h in gathers:
            h.wait()
        writes = (
            pltpu.async_copy(p_v, out_pos.at[rows], wsem),
            pltpu.async_copy(c_v, out_cat.at[rows], wsem),
            pltpu.async_copy(s_v, out_sense.at[rows], wsem),
            pltpu.async_copy(h_v, out_head.at[rows], wsem),
        )
        for h in writes:
            h.wait()


_gather = functools.partial(
    pl.kernel,
    mesh=plsc.VectorSubcoreMesh(core_axis_name="c", subcore_axis_name="s"),
    out_type=(
        jax.ShapeDtypeStruct((TOTAL, PAIR), jnp.float32),
        jax.ShapeDtypeStruct((TOTAL, PAIR), jnp.float32),
        jax.ShapeDtypeStruct((TOTAL, PAIR), jnp.float32),
        jax.ShapeDtypeStruct((TOTAL, ENC_SIZE), jnp.float32),
    ),
    scratch_types=[
        pltpu.VMEM((16, CHUNK), jnp.int32),
        pltpu.VMEM((CHUNK, PAIR), jnp.float32),
        pltpu.VMEM((CHUNK, PAIR), jnp.float32),
        pltpu.VMEM((CHUNK, PAIR), jnp.float32),
        pltpu.VMEM((CHUNK, ENC_SIZE), jnp.float32),
        pltpu.SemaphoreType.DMA,
        pltpu.SemaphoreType.DMA,
    ],
)(_gather_body)


def _half(g, par):
    lo = g[:, :EMB_DIM]
    hi = g[:, EMB_DIM:]
    return jnp.where(par > 0.5, hi, lo)


def _mm_body(gp_ref, gc_ref, gs_ref, h_ref, pp_ref, pc_ref, ps_ref,
             wp_ref, wc_ref, ws_ref, wh_ref, b_ref, o_ref):
    acc = jnp.dot(h_ref[...], wh_ref[...], preferred_element_type=jnp.float32)
    acc += jnp.dot(_half(gp_ref[...], pp_ref[...]), wp_ref[...],
                   preferred_element_type=jnp.float32)
    acc += jnp.dot(_half(gc_ref[...], pc_ref[...]), wc_ref[...],
                   preferred_element_type=jnp.float32)
    acc += jnp.dot(_half(gs_ref[...], ps_ref[...]), ws_ref[...],
                   preferred_element_type=jnp.float32)
    o_ref[...] = jnp.maximum(acc + b_ref[...], 0.0)


BM = 1024


def _matmul(gp, gc, gs, h, pp, pc, ps, wp, wc, ws, wh, b2d):
    pair_spec = pl.BlockSpec((BM, PAIR), lambda i: (i, 0))
    par_spec = pl.BlockSpec((BM, 1), lambda i: (i, 0))
    w_spec = pl.BlockSpec((EMB_DIM, REL_DIM), lambda i: (0, 0))
    return pl.pallas_call(
        _mm_body,
        grid=(TOTAL // BM,),
        in_specs=[
            pair_spec, pair_spec, pair_spec,
            pl.BlockSpec((BM, ENC_SIZE), lambda i: (i, 0)),
            par_spec, par_spec, par_spec,
            w_spec, w_spec, w_spec,
            pl.BlockSpec((ENC_SIZE, REL_DIM), lambda i: (0, 0)),
            pl.BlockSpec((1, REL_DIM), lambda i: (0, 0)),
        ],
        out_specs=pl.BlockSpec((BM, REL_DIM), lambda i: (i, 0)),
        out_shape=jax.ShapeDtypeStruct((TOTAL, REL_DIM), jnp.float32),
    )(gp, gc, gs, h, pp, pc, ps, wp, wc, ws, wh, b2d)


def kernel(input_data, index, src_enc_data, pos_table, cat_table, sense_table,
           W, b, lengths):
    ids_pos = input_data[:, 0].astype(jnp.int32)
    ids_cat = input_data[:, 1].astype(jnp.int32)
    ids_sense = input_data[:, 2].astype(jnp.int32)
    t = jnp.arange(TOTAL, dtype=jnp.int32)
    flat_idx = (t // SEQ_LEN) * SEQ_LEN + index.astype(jnp.int32)
    def pairize(tbl):
        return jnp.concatenate([tbl[0::2], tbl[1::2]], axis=1)

    pos2 = pairize(pos_table)
    cat2 = pairize(cat_table)
    sense2 = pairize(sense_table)
    pp = (ids_pos & 1).astype(jnp.float32).reshape(-1, 1)
    pc = (ids_cat & 1).astype(jnp.float32).reshape(-1, 1)
    ps = (ids_sense & 1).astype(jnp.float32).reshape(-1, 1)
    gp, gc, gs, h = _gather(ids_pos >> 1, ids_cat >> 1, ids_sense >> 1,
                            flat_idx, pos2, cat2, sense2, src_enc_data)
    wp = W[:EMB_DIM]
    wc = W[EMB_DIM:2 * EMB_DIM]
    ws = W[2 * EMB_DIM:3 * EMB_DIM]
    wh = W[3 * EMB_DIM:]
    return _matmul(gp, gc, gs, h, pp, pc, ps, wp, wc, ws, wh,
                   b.reshape(1, REL_DIM))


# speedup vs baseline: 7.3118x; 7.3118x over previous
import functools

import jax
import jax.numpy as jnp
from jax import lax
from jax.experimental import pallas as pl
from jax.experimental.pallas import tpu as pltpu
from jax.experimental.pallas import tpu_sc as plsc

BATCH = 16
SEQ_LEN = 1024
TOTAL = BATCH * SEQ_LEN
EMB_DIM = 64
ENC_SIZE = 512
REL_DIM = 256
PAIR = 2 * EMB_DIM

NUM_WORKERS = 32
TPW = TOTAL // NUM_WORKERS
CHUNK = 128
NCHUNK = TPW // CHUNK


def _gather_body(ids_pos, ids_cat, ids_sense, flat_idx,
                 pos_t, cat_t, sense_t, head_t,
                 out_pos, out_cat, out_sense, out_head,
                 idx_v, p_v, c_v, s_v, h_v, gsem, wsem):
    wid = lax.axis_index("s") * 2 + lax.axis_index("c")
    base = wid * TPW
    staged = []
    for k, ids in enumerate((ids_pos, ids_cat, ids_sense, flat_idx)):
        for j in range(NCHUNK):
            rows = pl.ds(base + j * CHUNK, CHUNK)
            staged.append(
                pltpu.async_copy(ids.at[rows], idx_v.at[k * NCHUNK + j], gsem))
    for h in staged:
        h.wait()
    for j in range(NCHUNK):
        rows = pl.ds(base + j * CHUNK, CHUNK)
        gathers = (
            pltpu.async_copy(pos_t.at[idx_v.at[0 * NCHUNK + j]], p_v, gsem),
            pltpu.async_copy(cat_t.at[idx_v.at[1 * NCHUNK + j]], c_v, gsem),
            pltpu.async_copy(sense_t.at[idx_v.at[2 * NCHUNK + j]], s_v, gsem),
            pltpu.async_copy(head_t.at[idx_v.at[3 * NCHUNK + j]], h_v, gsem),
        )
        for h in gathers:
            h.wait()
        writes = (
            pltpu.async_copy(p_v, out_pos.at[rows], wsem),
            pltpu.async_copy(c_v, out_cat.at[rows], wsem),
            pltpu.async_copy(s_v, out_sense.at[rows], wsem),
            pltpu.async_copy(h_v, out_head.at[rows], wsem),
        )
        for h in writes:
            h.wait()


_gather = functools.partial(
    pl.kernel,
    mesh=plsc.VectorSubcoreMesh(core_axis_name="c", subcore_axis_name="s"),
    out_type=(
        jax.ShapeDtypeStruct((TOTAL, PAIR), jnp.float32),
        jax.ShapeDtypeStruct((TOTAL, PAIR), jnp.float32),
        jax.ShapeDtypeStruct((TOTAL, PAIR), jnp.float32),
        jax.ShapeDtypeStruct((TOTAL, REL_DIM), jnp.float32),
    ),
    scratch_types=[
        pltpu.VMEM((16, CHUNK), jnp.int32),
        pltpu.VMEM((CHUNK, PAIR), jnp.float32),
        pltpu.VMEM((CHUNK, PAIR), jnp.float32),
        pltpu.VMEM((CHUNK, PAIR), jnp.float32),
        pltpu.VMEM((CHUNK, REL_DIM), jnp.float32),
        pltpu.SemaphoreType.DMA,
        pltpu.SemaphoreType.DMA,
    ],
)(_gather_body)


BM = 1024


def _head_body(x_ref, w_ref, o_ref):
    o_ref[...] = jnp.dot(x_ref[...], w_ref[...],
                         preferred_element_type=jnp.float32)


def _head_proj(x, wh):
    return pl.pallas_call(
        _head_body,
        grid=(TOTAL // BM,),
        in_specs=[
            pl.BlockSpec((BM, ENC_SIZE), lambda i: (i, 0)),
            pl.BlockSpec((ENC_SIZE, REL_DIM), lambda i: (0, 0)),
        ],
        out_specs=pl.BlockSpec((BM, REL_DIM), lambda i: (i, 0)),
        out_shape=jax.ShapeDtypeStruct((TOTAL, REL_DIM), jnp.float32),
    )(x, wh)


def _half(g, par):
    lo = g[:, :EMB_DIM]
    hi = g[:, EMB_DIM:]
    return jnp.where(par > 0.5, hi, lo)


def _mm_body(gp_ref, gc_ref, gs_ref, gh_ref, pp_ref, pc_ref, ps_ref,
             wp_ref, wc_ref, ws_ref, b_ref, o_ref):
    acc = gh_ref[...] + b_ref[...]
    acc += jnp.dot(_half(gp_ref[...], pp_ref[...]), wp_ref[...],
                   preferred_element_type=jnp.float32)
    acc += jnp.dot(_half(gc_ref[...], pc_ref[...]), wc_ref[...],
                   preferred_element_type=jnp.float32)
    acc += jnp.dot(_half(gs_ref[...], ps_ref[...]), ws_ref[...],
                   preferred_element_type=jnp.float32)
    o_ref[...] = jnp.maximum(acc, 0.0)


def _matmul(gp, gc, gs, gh, pp, pc, ps, wp, wc, ws, b2d):
    pair_spec = pl.BlockSpec((BM, PAIR), lambda i: (i, 0))
    par_spec = pl.BlockSpec((BM, 1), lambda i: (i, 0))
    w_spec = pl.BlockSpec((EMB_DIM, REL_DIM), lambda i: (0, 0))
    return pl.pallas_call(
        _mm_body,
        grid=(TOTAL // BM,),
        in_specs=[
            pair_spec, pair_spec, pair_spec,
            pl.BlockSpec((BM, REL_DIM), lambda i: (i, 0)),
            par_spec, par_spec, par_spec,
            w_spec, w_spec, w_spec,
            pl.BlockSpec((1, REL_DIM), lambda i: (0, 0)),
        ],
        out_specs=pl.BlockSpec((BM, REL_DIM), lambda i: (i, 0)),
        out_shape=jax.ShapeDtypeStruct((TOTAL, REL_DIM), jnp.float32),
    )(gp, gc, gs, gh, pp, pc, ps, wp, wc, ws, b2d)


def kernel(input_data, index, src_enc_data, pos_table, cat_table, sense_table,
           W, b, lengths):
    ids_pos = input_data[:, 0].astype(jnp.int32)
    ids_cat = input_data[:, 1].astype(jnp.int32)
    ids_sense = input_data[:, 2].astype(jnp.int32)
    t = jnp.arange(TOTAL, dtype=jnp.int32)
    flat_idx = (t // SEQ_LEN) * SEQ_LEN + index.astype(jnp.int32)
    pos2 = pos_table.reshape(-1, PAIR)
    cat2 = cat_table.reshape(-1, PAIR)
    sense2 = sense_table.reshape(-1, PAIR)
    pp = (ids_pos & 1).astype(jnp.float32).reshape(-1, 1)
    pc = (ids_cat & 1).astype(jnp.float32).reshape(-1, 1)
    ps = (ids_sense & 1).astype(jnp.float32).reshape(-1, 1)
    wp = W[:EMB_DIM]
    wc = W[EMB_DIM:2 * EMB_DIM]
    ws = W[2 * EMB_DIM:3 * EMB_DIM]
    wh = W[3 * EMB_DIM:]
    hproj = _head_proj(src_enc_data, wh)
    gp, gc, gs, gh = _gather(ids_pos >> 1, ids_cat >> 1, ids_sense >> 1,
                             flat_idx, pos2, cat2, sense2, hproj)
    return _matmul(gp, gc, gs, gh, pp, pc, ps, wp, wc, ws,
                   b.reshape(1, REL_DIM))


# speedup vs baseline: 8.6273x; 1.1799x over previous
import functools

import jax
import jax.numpy as jnp
from jax import lax
from jax.experimental import pallas as pl
from jax.experimental.pallas import tpu as pltpu
from jax.experimental.pallas import tpu_sc as plsc

BATCH = 16
SEQ_LEN = 1024
TOTAL = BATCH * SEQ_LEN
EMB_DIM = 64
ENC_SIZE = 512
REL_DIM = 256
PAIR = 2 * EMB_DIM

NUM_WORKERS = 32
TPW = TOTAL // NUM_WORKERS
CHUNK = 128
NCHUNK = TPW // CHUNK


def _gather_body(ids_pos, ids_cat, ids_sense, flat_idx,
                 pos_t, cat_t, sense_t, head_t,
                 out_pos, out_cat, out_sense, out_head,
                 idx_v, p_v, c_v, s_v, h_v, gsem, wsem):
    wid = lax.axis_index("s") * 2 + lax.axis_index("c")
    base = wid * TPW
    staged = []
    for k, ids in enumerate((ids_pos, ids_cat, ids_sense, flat_idx)):
        for j in range(NCHUNK):
            rows = pl.ds(base + j * CHUNK, CHUNK)
            staged.append(
                pltpu.async_copy(ids.at[rows], idx_v.at[k * NCHUNK + j], gsem))
    for h in staged:
        h.wait()
    for j in range(NCHUNK):
        rows = pl.ds(base + j * CHUNK, CHUNK)
        gathers = (
            pltpu.async_copy(pos_t.at[idx_v.at[0 * NCHUNK + j]], p_v, gsem),
            pltpu.async_copy(cat_t.at[idx_v.at[1 * NCHUNK + j]], c_v, gsem),
            pltpu.async_copy(sense_t.at[idx_v.at[2 * NCHUNK + j]], s_v, gsem),
            pltpu.async_copy(head_t.at[idx_v.at[3 * NCHUNK + j]], h_v, gsem),
        )
        for h in gathers:
            h.wait()
        writes = (
            pltpu.async_copy(p_v, out_pos.at[rows], wsem),
            pltpu.async_copy(c_v, out_cat.at[rows], wsem),
            pltpu.async_copy(s_v, out_sense.at[rows], wsem),
            pltpu.async_copy(h_v, out_head.at[rows], wsem),
        )
        for h in writes:
            h.wait()


_gather = functools.partial(
    pl.kernel,
    mesh=plsc.VectorSubcoreMesh(core_axis_name="c", subcore_axis_name="s"),
    out_type=(
        jax.ShapeDtypeStruct((TOTAL, PAIR), jnp.float32),
        jax.ShapeDtypeStruct((TOTAL, PAIR), jnp.float32),
        jax.ShapeDtypeStruct((TOTAL, PAIR), jnp.float32),
        jax.ShapeDtypeStruct((TOTAL, REL_DIM), jnp.float32),
    ),
    scratch_types=[
        pltpu.VMEM((16, CHUNK), jnp.int32),
        pltpu.VMEM((CHUNK, PAIR), jnp.float32),
        pltpu.VMEM((CHUNK, PAIR), jnp.float32),
        pltpu.VMEM((CHUNK, PAIR), jnp.float32),
        pltpu.VMEM((CHUNK, REL_DIM), jnp.float32),
        pltpu.SemaphoreType.DMA,
        pltpu.SemaphoreType.DMA,
    ],
)(_gather_body)


BM = 1024


def _head_body(x_ref, w_ref, o_ref):
    o_ref[...] = jnp.dot(x_ref[...], w_ref[...],
                         preferred_element_type=jnp.float32)


def _head_proj(x, wh):
    return pl.pallas_call(
        _head_body,
        grid=(TOTAL // BM,),
        in_specs=[
            pl.BlockSpec((BM, ENC_SIZE), lambda i: (i, 0)),
            pl.BlockSpec((ENC_SIZE, REL_DIM), lambda i: (0, 0)),
        ],
        out_specs=pl.BlockSpec((BM, REL_DIM), lambda i: (i, 0)),
        out_shape=jax.ShapeDtypeStruct((TOTAL, REL_DIM), jnp.float32),
    )(x, wh)


def _half(g, par):
    lo = g[:, :EMB_DIM]
    hi = g[:, EMB_DIM:]
    return jnp.where(par == 1, hi, lo)


def _mm_body(gp_ref, gc_ref, gs_ref, gh_ref, ids_ref,
             wp_ref, wc_ref, ws_ref, b_ref, o_ref):
    ids = ids_ref[...]
    acc = gh_ref[...] + b_ref[...]
    acc += jnp.dot(_half(gp_ref[...], ids[:, 0:1] & 1), wp_ref[...],
                   preferred_element_type=jnp.float32)
    acc += jnp.dot(_half(gc_ref[...], ids[:, 1:2] & 1), wc_ref[...],
                   preferred_element_type=jnp.float32)
    acc += jnp.dot(_half(gs_ref[...], ids[:, 2:3] & 1), ws_ref[...],
                   preferred_element_type=jnp.float32)
    o_ref[...] = jnp.maximum(acc, 0.0)


def _matmul(gp, gc, gs, gh, ids3, wp, wc, ws, b2d):
    pair_spec = pl.BlockSpec((BM, PAIR), lambda i: (i, 0))
    w_spec = pl.BlockSpec((EMB_DIM, REL_DIM), lambda i: (0, 0))
    return pl.pallas_call(
        _mm_body,
        grid=(TOTAL // BM,),
        in_specs=[
            pair_spec, pair_spec, pair_spec,
            pl.BlockSpec((BM, REL_DIM), lambda i: (i, 0)),
            pl.BlockSpec((BM, 3), lambda i: (i, 0)),
            w_spec, w_spec, w_spec,
            pl.BlockSpec((1, REL_DIM), lambda i: (0, 0)),
        ],
        out_specs=pl.BlockSpec((BM, REL_DIM), lambda i: (i, 0)),
        out_shape=jax.ShapeDtypeStruct((TOTAL, REL_DIM), jnp.float32),
    )(gp, gc, gs, gh, ids3, wp, wc, ws, b2d)


def kernel(input_data, index, src_enc_data, pos_table, cat_table, sense_table,
           W, b, lengths):
    ids_pos = input_data[:, 0].astype(jnp.int32)
    ids_cat = input_data[:, 1].astype(jnp.int32)
    ids_sense = input_data[:, 2].astype(jnp.int32)
    t = jnp.arange(TOTAL, dtype=jnp.int32)
    flat_idx = (t // SEQ_LEN) * SEQ_LEN + index.astype(jnp.int32)
    pos2 = pos_table.reshape(-1, PAIR)
    cat2 = cat_table.reshape(-1, PAIR)
    sense2 = sense_table.reshape(-1, PAIR)
    wp = W[:EMB_DIM]
    wc = W[EMB_DIM:2 * EMB_DIM]
    ws = W[2 * EMB_DIM:3 * EMB_DIM]
    wh = W[3 * EMB_DIM:]
    hproj = _head_proj(src_enc_data, wh)
    gp, gc, gs, gh = _gather(ids_pos >> 1, ids_cat >> 1, ids_sense >> 1,
                             flat_idx, pos2, cat2, sense2, hproj)
    return _matmul(gp, gc, gs, gh, input_data.astype(jnp.int32),
                   wp, wc, ws, b.reshape(1, REL_DIM))
